# baseline (speedup 1.0000x reference)
import functools

import jax
import jax.numpy as jnp
from jax import lax
from jax.experimental import pallas as pl
from jax.experimental.pallas import tpu as pltpu

_SZ = 32
_BIMG = 64
_G = 4


def _conv3x3_relu(a3, wcat, bias, w):
    g, hw, kb = a3.shape
    nb = wcat.shape[1] // 3
    zp = jnp.zeros((g, w, kb), a3.dtype)
    up = jnp.concatenate([zp, a3[:, :hw - w, :]], axis=1)
    dn = jnp.concatenate([a3[:, w:, :], zp], axis=1)
    cy = jnp.concatenate([up, a3, dn], axis=2)

    z = jnp.dot(cy.reshape(g * hw, 3 * kb), wcat,
                preferred_element_type=jnp.float32).astype(jnp.bfloat16)

    m = g * hw
    xo = lax.broadcasted_iota(jnp.int32, (m, nb), 0) & (w - 1)
    zb = jnp.zeros((), jnp.bfloat16)
    left = jnp.where(xo != 0, jnp.pad(z[:, :nb], ((1, 0), (0, 0)))[:m], zb)
    right = jnp.where(xo != w - 1,
                      jnp.pad(z[:, 2 * nb:], ((0, 1), (0, 0)))[1:], zb)
    y = z[:, nb:2 * nb] + left + right + bias
    return jnp.maximum(y, zb).reshape(g, hw, nb)


def _pool2x2(a3, w):
    g, hw, c = a3.shape
    m = g * hw
    v = a3.reshape(m // (2 * w), 2, w, c)
    t = jnp.maximum(v[:, 0], v[:, 1])
    v2 = t.reshape(m // 4, 2, c)
    return jnp.maximum(v2[:, 0], v2[:, 1]).reshape(g, hw // 4, c)


def _tower_kernel(x_ref, wc1, bb1, wc2, bb2, wc3, bb3, wc4, bb4,
                  wc5, bb5, wc6, bb6, o_ref, *, bimg, size):
    h1, h2, h3 = size, size // 2, size // 4
    at = jnp.transpose(x_ref[...], (0, 2, 1)).astype(jnp.bfloat16)
    xg = at.reshape(bimg // _G, _G, h1 * h1, 3)
    a = jnp.concatenate([xg[:, i] for i in range(_G)], axis=2)
    a = jnp.pad(a, ((0, 0), (0, 0), (0, 128 - a.shape[2])))

    a = _conv3x3_relu(a, wc1[...], bb1[...], h1)
    a = _conv3x3_relu(a, wc2[...], bb2[...], h1)
    a = _pool2x2(a, h1)
    a = _conv3x3_relu(a, wc3[...], bb3[...], h2)
    a = _conv3x3_relu(a, wc4[...], bb4[...], h2)
    a = _pool2x2(a, h2)
    a = _conv3x3_relu(a, wc5[...], bb5[...], h3)
    a = _conv3x3_relu(a, wc6[...], bb6[...], h3)
    a = _pool2x2(a, h3)
    o_ref[...] = a.astype(o_ref.dtype)


def _fc_kernel(f_ref, w1_ref, b1_ref, wh_ref, bh_ref, o_ref):
    h = jnp.dot(f_ref[...], w1_ref[...],
                preferred_element_type=jnp.float32) + b1_ref[...]
    h = jnp.maximum(h, 0.0).astype(jnp.bfloat16)
    o_ref[...] = jnp.dot(h, wh_ref[...],
                         preferred_element_type=jnp.float32) + bh_ref[...]


def _pack_conv(w9, kb1=None):
    cin, cout = w9.shape[1], w9.shape[2]
    w9r = w9.reshape(3, 3, cin, cout)
    eye = jnp.eye(_G, dtype=w9.dtype)
    t = jnp.einsum("kxco,ij->kicxjo", w9r, eye)
    t = t.reshape(3, _G * cin, 3 * _G * cout)
    if kb1 is not None:
        t = jnp.pad(t, ((0, 0), (0, kb1 - _G * cin), (0, 0)))
    return t.reshape(-1, 3 * _G * cout).astype(jnp.bfloat16)


def _pack_bias(b):
    return jnp.tile(b, (1, _G)).astype(jnp.bfloat16)


def _block_diag4(wm):
    k, nn = wm.shape
    eye = jnp.eye(_G, dtype=wm.dtype)
    t = jnp.einsum("kn,ij->ikjn", wm, eye)
    return t.reshape(_G * k, _G * nn).astype(jnp.bfloat16)


def kernel(x, w9_1, b_1, w9_2, b_2, w9_3, b_3, w9_4, b_4, w9_5, b_5,
           w9_6, b_6, w1, b1, wh, bh, s1, s2, s3):
    del s1, s2, s3
    n = x.shape[0]
    size = _SZ
    sf = size // 8
    ss = sf * sf
    bimg = _BIMG

    x_flat = x.reshape(n, 3, size * size)

    wcs = [_pack_conv(w9_1, kb1=128)] + [
        _pack_conv(w) for w in (w9_2, w9_3, w9_4, w9_5, w9_6)]
    bbs = [_pack_bias(b) for b in (b_1, b_2, b_3, b_4, b_5, b_6)]

    def const_spec(shape):
        zeros = (0,) * len(shape)
        return pl.BlockSpec(shape, lambda i, _z=zeros: _z)

    in_specs = [pl.BlockSpec((bimg, 3, size * size), lambda i: (i, 0, 0))]
    args = [x_flat]
    for wc, bb in zip(wcs, bbs):
        in_specs += [const_spec(wc.shape), const_spec(bb.shape)]
        args += [wc, bb]

    feat = pl.pallas_call(
        functools.partial(_tower_kernel, bimg=bimg, size=size),
        out_shape=jax.ShapeDtypeStruct((n // _G, ss, _G * 128), jnp.bfloat16),
        grid=(n // bimg,),
        in_specs=in_specs,
        out_specs=pl.BlockSpec((bimg // _G, ss, _G * 128), lambda i: (i, 0, 0)),
        compiler_params=pltpu.CompilerParams(
            dimension_semantics=("parallel",),
            vmem_limit_bytes=100 * 1024 * 1024),
    )(*args)

    feat2 = feat.reshape(n // _G, ss * _G * 128)
    w1r = w1.reshape(ss, 128, w1.shape[1])
    eye = jnp.eye(_G, dtype=w1.dtype)
    w1p = jnp.einsum("pch,ij->picjh", w1r, eye).reshape(
        ss * _G * 128, _G * w1.shape[1]).astype(jnp.bfloat16)
    b1p = jnp.tile(b1, (1, _G))
    whp = _block_diag4(wh)
    bhp = jnp.tile(bh, (1, _G))

    hid = _G * w1.shape[1]
    npad = _G * wh.shape[1]
    nrow = n // _G
    nblk = nrow // 2
    y_all = pl.pallas_call(
        _fc_kernel,
        out_shape=jax.ShapeDtypeStruct((nrow, npad), jnp.float32),
        grid=(2,),
        in_specs=[
            pl.BlockSpec((nblk, ss * _G * 128), lambda i: (i, 0)),
            pl.BlockSpec((ss * _G * 128, hid), lambda i: (0, 0)),
            pl.BlockSpec((1, hid), lambda i: (0, 0)),
            pl.BlockSpec((hid, npad), lambda i: (0, 0)),
            pl.BlockSpec((1, npad), lambda i: (0, 0)),
        ],
        out_specs=pl.BlockSpec((nblk, npad), lambda i: (i, 0)),
        compiler_params=pltpu.CompilerParams(
            dimension_semantics=("parallel",),
            vmem_limit_bytes=64 * 1024 * 1024),
    )(feat2, w1p, b1p, whp, bhp)

    y2 = y_all.reshape(n, wh.shape[1])
    outs, off = [], 0
    for _ in range(10):
        outs.append(y2[:, off:off + 10])
        off += 10
    return outs


# speedup vs baseline: 1.1882x; 1.1882x over previous
import functools

import jax
import jax.numpy as jnp
from jax import lax
from jax.experimental import pallas as pl
from jax.experimental.pallas import tpu as pltpu

_SZ = 32
_BIMG = 32
_G = 4


def _conv3x3_relu(a3, wcat, bias, w):
    g, hw, kb = a3.shape
    nb = wcat.shape[1] // 3
    zp = jnp.zeros((g, w, kb), a3.dtype)
    up = jnp.concatenate([zp, a3[:, :hw - w, :]], axis=1)
    dn = jnp.concatenate([a3[:, w:, :], zp], axis=1)
    cy = jnp.concatenate([up, a3, dn], axis=2)

    z = jnp.dot(cy.reshape(g * hw, 3 * kb), wcat,
                preferred_element_type=jnp.float32).astype(jnp.bfloat16)

    m = g * hw
    xo = lax.broadcasted_iota(jnp.int32, (m, nb), 0) & (w - 1)
    zb = jnp.zeros((), jnp.bfloat16)
    left = jnp.where(xo != 0, jnp.pad(z[:, :nb], ((1, 0), (0, 0)))[:m], zb)
    right = jnp.where(xo != w - 1,
                      jnp.pad(z[:, 2 * nb:], ((0, 1), (0, 0)))[1:], zb)
    y = z[:, nb:2 * nb] + left + right + bias
    return jnp.maximum(y, zb).reshape(g, hw, nb)


def _pool2x2(a3, w):
    g, hw, c = a3.shape
    m = g * hw
    v = a3.reshape(m // (2 * w), 2, w, c)
    t = jnp.maximum(v[:, 0], v[:, 1])
    v2 = t.reshape(m // 4, 2, c)
    return jnp.maximum(v2[:, 0], v2[:, 1]).reshape(g, hw // 4, c)


def _tower_kernel(x_ref, wc1, bb1, wc2, bb2, wc3, bb3, wc4, bb4,
                  wc5, bb5, wc6, bb6, o_ref, *, bimg, size):
    h1, h2, h3 = size, size // 2, size // 4
    at = jnp.transpose(x_ref[...], (0, 2, 1)).astype(jnp.bfloat16)
    xg = at.reshape(bimg // _G, _G, h1 * h1, 3)
    a = jnp.concatenate([xg[:, i] for i in range(_G)], axis=2)
    a = jnp.pad(a, ((0, 0), (0, 0), (0, 128 - a.shape[2])))

    a = _conv3x3_relu(a, wc1[...], bb1[...], h1)
    a = _conv3x3_relu(a, wc2[...], bb2[...], h1)
    a = _pool2x2(a, h1)
    a = _conv3x3_relu(a, wc3[...], bb3[...], h2)
    a = _conv3x3_relu(a, wc4[...], bb4[...], h2)
    a = _pool2x2(a, h2)
    a = _conv3x3_relu(a, wc5[...], bb5[...], h3)
    a = _conv3x3_relu(a, wc6[...], bb6[...], h3)
    a = _pool2x2(a, h3)
    o_ref[...] = a.astype(o_ref.dtype)


def _fc_kernel(f_ref, w1_ref, b1_ref, wh_ref, bh_ref, o_ref):
    h = jnp.dot(f_ref[...], w1_ref[...],
                preferred_element_type=jnp.float32) + b1_ref[...]
    h = jnp.maximum(h, 0.0).astype(jnp.bfloat16)
    o_ref[...] = jnp.dot(h, wh_ref[...],
                         preferred_element_type=jnp.float32) + bh_ref[...]


def _pack_conv(w9, kb1=None):
    cin, cout = w9.shape[1], w9.shape[2]
    w9r = w9.reshape(3, 3, cin, cout)
    eye = jnp.eye(_G, dtype=w9.dtype)
    t = jnp.einsum("kxco,ij->kicxjo", w9r, eye)
    t = t.reshape(3, _G * cin, 3 * _G * cout)
    if kb1 is not None:
        t = jnp.pad(t, ((0, 0), (0, kb1 - _G * cin), (0, 0)))
    return t.reshape(-1, 3 * _G * cout).astype(jnp.bfloat16)


def _pack_bias(b):
    return jnp.tile(b, (1, _G)).astype(jnp.bfloat16)


def _block_diag4(wm):
    k, nn = wm.shape
    wmb = wm.astype(jnp.bfloat16)
    eye = jnp.eye(_G, dtype=jnp.bfloat16)
    t = jnp.einsum("kn,ij->ikjn", wmb, eye)
    return t.reshape(_G * k, _G * nn)


def kernel(x, w9_1, b_1, w9_2, b_2, w9_3, b_3, w9_4, b_4, w9_5, b_5,
           w9_6, b_6, w1, b1, wh, bh, s1, s2, s3):
    del s1, s2, s3
    n = x.shape[0]
    size = _SZ
    sf = size // 8
    ss = sf * sf
    bimg = _BIMG

    x_flat = x.reshape(n, 3, size * size)

    wcs = [_pack_conv(w9_1, kb1=128)] + [
        _pack_conv(w) for w in (w9_2, w9_3, w9_4, w9_5, w9_6)]
    bbs = [_pack_bias(b) for b in (b_1, b_2, b_3, b_4, b_5, b_6)]

    def const_spec(shape):
        zeros = (0,) * len(shape)
        return pl.BlockSpec(shape, lambda i, _z=zeros: _z)

    in_specs = [pl.BlockSpec((bimg, 3, size * size), lambda i: (i, 0, 0))]
    args = [x_flat]
    for wc, bb in zip(wcs, bbs):
        in_specs += [const_spec(wc.shape), const_spec(bb.shape)]
        args += [wc, bb]

    feat = pl.pallas_call(
        functools.partial(_tower_kernel, bimg=bimg, size=size),
        out_shape=jax.ShapeDtypeStruct((n // _G, ss, _G * 128), jnp.bfloat16),
        grid=(n // bimg,),
        in_specs=in_specs,
        out_specs=pl.BlockSpec((bimg // _G, ss, _G * 128), lambda i: (i, 0, 0)),
        compiler_params=pltpu.CompilerParams(
            dimension_semantics=("parallel",),
            vmem_limit_bytes=100 * 1024 * 1024),
    )(*args)

    feat2 = feat.reshape(n // _G, ss * _G * 128)
    w1r = w1.reshape(ss, 128, w1.shape[1]).astype(jnp.bfloat16)
    eye = jnp.eye(_G, dtype=jnp.bfloat16)
    w1p = jnp.einsum("pch,ij->picjh", w1r, eye).reshape(
        ss * _G * 128, _G * w1.shape[1])
    b1p = jnp.tile(b1, (1, _G))
    whp = _block_diag4(wh)
    bhp = jnp.tile(bh, (1, _G))

    hid = _G * w1.shape[1]
    npad = _G * wh.shape[1]
    nrow = n // _G
    nblk = nrow // 2
    y_all = pl.pallas_call(
        _fc_kernel,
        out_shape=jax.ShapeDtypeStruct((nrow, npad), jnp.float32),
        grid=(2,),
        in_specs=[
            pl.BlockSpec((nblk, ss * _G * 128), lambda i: (i, 0)),
            pl.BlockSpec((ss * _G * 128, hid), lambda i: (0, 0)),
            pl.BlockSpec((1, hid), lambda i: (0, 0)),
            pl.BlockSpec((hid, npad), lambda i: (0, 0)),
            pl.BlockSpec((1, npad), lambda i: (0, 0)),
        ],
        out_specs=pl.BlockSpec((nblk, npad), lambda i: (i, 0)),
        compiler_params=pltpu.CompilerParams(
            dimension_semantics=("parallel",),
            vmem_limit_bytes=64 * 1024 * 1024),
    )(feat2, w1p, b1p, whp, bhp)

    y2 = y_all.reshape(n, wh.shape[1])
    outs, off = [], 0
    for _ in range(10):
        outs.append(y2[:, off:off + 10])
        off += 10
    return outs
